# baseline (speedup 1.0000x reference)
import jax
import jax.numpy as jnp
from jax.experimental import pallas as pl
from jax.experimental.pallas import tpu as pltpu

TC_BS = 512


def _tc_add_body(x_ref, pe_ref, o_ref):
    o_ref[...] = x_ref[...] + pe_ref[...]


def _tc_add(x, pe_table):
    B, S, D = x.shape
    return pl.pallas_call(
        _tc_add_body,
        grid=(S // TC_BS, B),
        in_specs=[
            pl.BlockSpec((1, TC_BS, D), lambda i, b: (b, i, 0)),
            pl.BlockSpec((TC_BS, D), lambda i, b: (i, 0)),
        ],
        out_specs=pl.BlockSpec((1, TC_BS, D), lambda i, b: (b, i, 0)),
        out_shape=jax.ShapeDtypeStruct((B, S, D), x.dtype),
    )(x, pe_table)


def kernel(x, pe_table):
    out_a = _tc_add(x[:3], pe_table)
    out_b = _tc_add(x[3:], pe_table)
    return jnp.concatenate([out_a, out_b], axis=0)


# speedup vs baseline: 2.9291x; 2.9291x over previous
import jax
import jax.numpy as jnp
from jax.experimental import pallas as pl

TC_BS = 512


def _tc_add_body(x_ref, pe_ref, o_ref):
    o_ref[...] = x_ref[...] + pe_ref[...]


def kernel(x, pe_table):
    B, S, D = x.shape
    return pl.pallas_call(
        _tc_add_body,
        grid=(S // TC_BS, B),
        in_specs=[
            pl.BlockSpec((1, TC_BS, D), lambda i, b: (b, i, 0)),
            pl.BlockSpec((TC_BS, D), lambda i, b: (i, 0)),
        ],
        out_specs=pl.BlockSpec((1, TC_BS, D), lambda i, b: (b, i, 0)),
        out_shape=jax.ShapeDtypeStruct((B, S, D), x.dtype),
    )(x, pe_table)
